# baseline (speedup 1.0000x reference)
import jax
import jax.numpy as jnp
from jax import lax
from jax.experimental import pallas as pl
from jax.experimental.pallas import tpu as pltpu
from jax.experimental.pallas import tpu_sc as plsc

NC, NS, L = 2, 16, 16
D, H, W = 10, 26, 26
P = D * H * W
B, COUT = 8, 128
PPW = 212
CHUNKS = (52, 52, 52, 56)
NBLK = COUT // L

ROW_BASE = H * 44
DEP_BASE = 2 * H * 44
TBL_LEN = DEP_BASE + D * 44


def _pos_body(tbl_hbm, out_hbm, tbl, src, sem):
    wid = lax.axis_index("s") * NC + lax.axis_index("c")
    p0 = jnp.minimum(wid * PPW, P - PPW)

    pltpu.sync_copy(tbl_hbm, tbl)

    step2 = jnp.where(lax.iota(jnp.int32, L) < 12, 44, 0)

    def row_body(r, idxs):
        p = p0 + r
        i = lax.rem(p, W)

        def recompute(_):
            j = lax.rem(lax.div(p, W), H)
            k = lax.div(p, H * W)
            oi = 44 * i
            oj = ROW_BASE - 44 + 44 * j
            ok = DEP_BASE - 88 + 44 * k
            out = []
            for m in range(NBLK):
                c = lax.iota(jnp.int32, L) + (L * m)
                if m < 2:
                    off = jnp.full((L,), oi, jnp.int32)
                elif m == 2:
                    off = jnp.where(c < 44, oi, oj)
                elif m < 5:
                    off = jnp.full((L,), oj, jnp.int32)
                elif m == 5:
                    off = jnp.where(c < 88, oj, ok)
                else:
                    off = jnp.full((L,), ok, jnp.int32)
                out.append(c + off)
            return tuple(out)

        def advance(cur):
            return (cur[0] + 44, cur[1] + 44, cur[2] + step2) + cur[3:]

        idxs = lax.cond((i == 0) | (r == 0), recompute, advance, idxs)
        for m in range(NBLK):
            src[r, pl.ds(L * m, L)] = plsc.load_gather(tbl, [idxs[m]])
        return idxs

    zero = jnp.zeros((L,), jnp.int32)
    carry = (zero,) * NBLK
    copies = []
    base = 0
    for cnt in CHUNKS:
        carry = lax.fori_loop(base, base + cnt, row_body, carry)
        for b in range(B):
            copies.append(
                pltpu.async_copy(
                    src.at[pl.ds(base, cnt)],
                    out_hbm.at[pl.ds(p0 + base, cnt), b],
                    sem,
                )
            )
        base += cnt
    for cp in copies:
        cp.wait()


@jax.jit
def _pos_embed(row_w, col_w, depth_w):
    mesh = plsc.VectorSubcoreMesh(
        core_axis_name="c", subcore_axis_name="s", num_cores=NC, num_subcores=NS
    )
    k = pl.kernel(
        _pos_body,
        out_type=jax.ShapeDtypeStruct((P, B, COUT), jnp.float32),
        mesh=mesh,
        compiler_params=pltpu.CompilerParams(needs_layout_passes=False),
        scratch_types=[
            pltpu.VMEM((TBL_LEN,), jnp.float32),
            pltpu.VMEM((PPW, COUT), jnp.float32),
            pltpu.SemaphoreType.DMA,
        ],
    )
    cat = jnp.concatenate([col_w, row_w, depth_w], axis=0).reshape(-1)
    return k(cat)


def kernel(x, row_w, col_w, depth_w):
    out = _pos_embed(row_w, col_w, depth_w)
    return out.reshape(D, H, W, B, COUT).transpose(3, 4, 0, 1, 2)


# speedup vs baseline: 1.0057x; 1.0057x over previous
import jax
import jax.numpy as jnp
from jax import lax
from jax.experimental import pallas as pl
from jax.experimental.pallas import tpu as pltpu
from jax.experimental.pallas import tpu_sc as plsc

NC, NS, L = 2, 16, 16
D, H, W = 10, 26, 26
P = D * H * W
B, COUT = 8, 128
PPW = 212
CHUNKS = (52, 52, 52, 56)
NBLK = COUT // L

ROW_BASE = H * 44
DEP_BASE = 2 * H * 44
TBL_LEN = DEP_BASE + D * 44


def _pos_body(tbl_hbm, out_hbm, tbl, src, sem):
    wid = lax.axis_index("s") * NC + lax.axis_index("c")
    p0 = jnp.minimum(wid * PPW, P - PPW)

    pltpu.sync_copy(tbl_hbm, tbl)

    def row_body(r, carry):
        p = p0 + r
        i = lax.rem(p, W)
        j = lax.rem(lax.div(p, W), H)
        k = lax.div(p, H * W)
        oi = 44 * i
        oj = ROW_BASE - 44 + 44 * j
        ok = DEP_BASE - 88 + 44 * k
        for m in range(NBLK):
            c = lax.iota(jnp.int32, L) + (L * m)
            if m < 2:
                off = jnp.full((L,), oi, jnp.int32)
            elif m == 2:
                off = jnp.where(c < 44, oi, oj)
            elif m < 5:
                off = jnp.full((L,), oj, jnp.int32)
            elif m == 5:
                off = jnp.where(c < 88, oj, ok)
            else:
                off = jnp.full((L,), ok, jnp.int32)
            src[r, pl.ds(L * m, L)] = plsc.load_gather(tbl, [c + off])
        return carry

    copies = []
    base = 0
    for cnt in CHUNKS:
        lax.fori_loop(base, base + cnt, row_body, 0)
        for b in range(B):
            copies.append(
                pltpu.async_copy(
                    src.at[pl.ds(base, cnt)],
                    out_hbm.at[pl.ds(p0 + base, cnt), b],
                    sem,
                )
            )
        base += cnt
    for cp in copies:
        cp.wait()


@jax.jit
def _pos_embed(row_w, col_w, depth_w):
    mesh = plsc.VectorSubcoreMesh(
        core_axis_name="c", subcore_axis_name="s", num_cores=NC, num_subcores=NS
    )
    k = pl.kernel(
        _pos_body,
        out_type=jax.ShapeDtypeStruct((P, B, COUT), jnp.float32),
        mesh=mesh,
        compiler_params=pltpu.CompilerParams(needs_layout_passes=False),
        scratch_types=[
            pltpu.VMEM((TBL_LEN,), jnp.float32),
            pltpu.VMEM((PPW, COUT), jnp.float32),
            pltpu.SemaphoreType.DMA,
        ],
    )
    cat = jnp.concatenate([col_w, row_w, depth_w], axis=0).reshape(-1)
    return k(cat)


def kernel(x, row_w, col_w, depth_w):
    out = _pos_embed(row_w, col_w, depth_w)
    return out.reshape(D, H, W, B, COUT).transpose(3, 4, 0, 1, 2)


# speedup vs baseline: 1.0063x; 1.0006x over previous
import jax
import jax.numpy as jnp
from jax import lax
from jax.experimental import pallas as pl
from jax.experimental.pallas import tpu as pltpu
from jax.experimental.pallas import tpu_sc as plsc

NC, NS, L = 2, 16, 16
D, H, W = 10, 26, 26
P = D * H * W
B, COUT = 8, 128
PPW = 212
CHUNKS = (28, 28, 28, 28, 28, 28, 28, 16)
NBLK = COUT // L

ROW_BASE = H * 44
DEP_BASE = 2 * H * 44
TBL_LEN = DEP_BASE + D * 44


def _pos_body(tbl_hbm, out_hbm, tbl, src, sem):
    wid = lax.axis_index("s") * NC + lax.axis_index("c")
    p0 = jnp.minimum(wid * PPW, P - PPW)

    pltpu.sync_copy(tbl_hbm, tbl)

    def row_body(r, carry):
        p = p0 + r
        i = lax.rem(p, W)
        j = lax.rem(lax.div(p, W), H)
        k = lax.div(p, H * W)
        oi = 44 * i
        oj = ROW_BASE - 44 + 44 * j
        ok = DEP_BASE - 88 + 44 * k
        for m in range(NBLK):
            c = lax.iota(jnp.int32, L) + (L * m)
            if m < 2:
                off = jnp.full((L,), oi, jnp.int32)
            elif m == 2:
                off = jnp.where(c < 44, oi, oj)
            elif m < 5:
                off = jnp.full((L,), oj, jnp.int32)
            elif m == 5:
                off = jnp.where(c < 88, oj, ok)
            else:
                off = jnp.full((L,), ok, jnp.int32)
            src[r, pl.ds(L * m, L)] = plsc.load_gather(tbl, [c + off])
        return carry

    copies = []
    base = 0
    for cnt in CHUNKS:
        lax.fori_loop(base, base + cnt, row_body, 0)
        for b in range(B):
            copies.append(
                pltpu.async_copy(
                    src.at[pl.ds(base, cnt)],
                    out_hbm.at[pl.ds(p0 + base, cnt), b],
                    sem,
                )
            )
        base += cnt
    for cp in copies:
        cp.wait()


@jax.jit
def _pos_embed(row_w, col_w, depth_w):
    mesh = plsc.VectorSubcoreMesh(
        core_axis_name="c", subcore_axis_name="s", num_cores=NC, num_subcores=NS
    )
    k = pl.kernel(
        _pos_body,
        out_type=jax.ShapeDtypeStruct((P, B, COUT), jnp.float32),
        mesh=mesh,
        compiler_params=pltpu.CompilerParams(needs_layout_passes=False),
        scratch_types=[
            pltpu.VMEM((TBL_LEN,), jnp.float32),
            pltpu.VMEM((PPW, COUT), jnp.float32),
            pltpu.SemaphoreType.DMA,
        ],
    )
    cat = jnp.concatenate([col_w, row_w, depth_w], axis=0).reshape(-1)
    return k(cat)


def kernel(x, row_w, col_w, depth_w):
    out = _pos_embed(row_w, col_w, depth_w)
    return out.reshape(D, H, W, B, COUT).transpose(3, 4, 0, 1, 2)
